# baseline (speedup 1.0000x reference)
import functools

import jax
import jax.numpy as jnp
from jax import lax
from jax.experimental import pallas as pl
from jax.experimental.pallas import tpu as pltpu
from jax.experimental.pallas import tpu_sc as plsc

N_NODES = 10000
N_EDGES = 320000
D = 128
NH = 8
NHP = 16

NC = 2
NS = 16
NW = NC * NS

EB3 = 1024
EB4 = 64
E_PER_W = 10240
E_PAD = NW * E_PER_W
N_PAD = 10112
ROWS_PER_TILE = N_PAD // NS
TRASH = N_PAD - 1


def _k1_body(nf_ref, wsT_ref, wdT_ref, wscT_ref, ps_ref, pd_ref, ss_ref, sd_ref):
    nf = nf_ref[...]
    ps = jnp.dot(nf, wsT_ref[...], preferred_element_type=jnp.float32)
    pd = jnp.dot(nf, wdT_ref[...], preferred_element_type=jnp.float32)
    ps_ref[...] = ps
    pd_ref[...] = pd
    ss_ref[...] = jnp.dot(ps, wscT_ref[...], preferred_element_type=jnp.float32)
    sd_ref[...] = jnp.dot(pd, wscT_ref[...], preferred_element_type=jnp.float32)


def _k2_body(ef_ref, weT_ref, bt_ref, wscT_ref, bs_ref, eh_ref, se_ref):
    bf = jnp.bfloat16
    eh = jnp.dot(ef_ref[...].astype(bf), weT_ref[...].astype(bf),
                 preferred_element_type=jnp.float32)
    eh = eh + bt_ref[...]
    eh_ref[...] = eh
    se_ref[...] = jnp.dot(eh.astype(bf), wscT_ref[...].astype(bf),
                          preferred_element_type=jnp.float32) + bs_ref[...]


def _k3b_body(d0_ref, d1_ref, out_ref):
    out_ref[...] = d0_ref[...] + d1_ref[...]


def _k5_body(nf_ref, wselfT_ref, bself_ref, m0_ref, m1_ref, den_ref,
             pd_ref, out_ref):
    self_msg = jnp.dot(nf_ref[...], wselfT_ref[...], preferred_element_type=jnp.float32)
    c = jnp.where(den_ref[...][:, 0:1] > 0, 1.0, 0.0)
    out = m0_ref[...] + m1_ref[...] + c * pd_ref[...] + self_msg + bself_ref[...]
    slope = (1.0 / 8.0 + 1.0 / 3.0) / 2.0
    out_ref[...] = jnp.where(out >= 0, out, slope * out)


def _leaky(v):
    return jnp.where(v >= 0, v, 0.01 * v)


def _k3_body(src2_hbm, dst2_hbm, se_hbm, ss_hbm, sd_hbm, zden_hbm,
             ex_hbm, den_hbm,
             sidx_v, didx_v, se_v, ssg_v, sdg_v, ex_v, den_sh,
             sem_i, sem_l, sem_g, sem_s):
    cid = lax.axis_index("c")
    sid = lax.axis_index("s")
    wid = sid * NC + cid

    r0 = pl.multiple_of(sid * ROWS_PER_TILE, 8)
    pltpu.sync_copy(zden_hbm.at[pl.ds(r0, ROWS_PER_TILE)], den_sh.at[pl.ds(r0, ROWS_PER_TILE)])
    plsc.subcore_barrier()

    nchunks = E_PER_W // EB3
    ngrp = EB3 // 128

    def chunk(c, _):
        e0 = pl.multiple_of(wid * E_PER_W + c * EB3, 8)
        row0 = pl.multiple_of(e0 // 128, 8)
        di0 = pltpu.async_copy(src2_hbm.at[pl.ds(row0, ngrp)], sidx_v, sem_i)
        di1 = pltpu.async_copy(dst2_hbm.at[pl.ds(row0, ngrp)], didx_v, sem_i)
        dl = pltpu.async_copy(se_hbm.at[pl.ds(e0, EB3)], se_v, sem_l)
        di0.wait()
        di1.wait()
        gs = []
        for g in range(ngrp):
            gs.append(pltpu.async_copy(ss_hbm.at[sidx_v.at[g]],
                                       ssg_v.at[pl.ds(g * 128, 128)], sem_g))
            gs.append(pltpu.async_copy(sd_hbm.at[didx_v.at[g]],
                                       sdg_v.at[pl.ds(g * 128, 128)], sem_g))
        dl.wait()
        for d in gs:
            d.wait()

        def edge(i, _):
            v = se_v[i] + ssg_v[i] + sdg_v[i]
            ex_v[i] = jnp.exp(_leaky(v))
            return 0

        lax.fori_loop(0, EB3, edge, 0)
        ws = [pltpu.async_copy(ex_v, ex_hbm.at[pl.ds(e0, EB3)], sem_l)]
        for g in range(ngrp):
            ws.append(pltpu.async_copy(ex_v.at[pl.ds(g * 128, 128)],
                                       den_sh.at[didx_v.at[g]], sem_s, add=True))
        for d in ws:
            d.wait()
        return 0

    lax.fori_loop(0, nchunks, chunk, 0)

    plsc.subcore_barrier()
    pltpu.sync_copy(den_sh.at[pl.ds(r0, ROWS_PER_TILE)],
                    den_hbm.at[cid].at[pl.ds(r0, ROWS_PER_TILE)])


def _k4_body(src2_hbm, dst2_hbm, ex_hbm, den_hbm, eh_hbm, ps_hbm,
             zmsg_hbm,
             msg_hbm,
             sidx_v, didx_v, ex_a, dn_a, eh_a, ps_a, ex_b, dn_b, eh_b, ps_b,
             w_v, msg_sh,
             sem_i, sem_la, sem_ga, sem_lb, sem_gb, sem_s):
    cid = lax.axis_index("c")
    sid = lax.axis_index("s")
    wid = sid * NC + cid

    r0 = pl.multiple_of(sid * ROWS_PER_TILE, 8)
    pltpu.sync_copy(zmsg_hbm.at[pl.ds(r0, ROWS_PER_TILE)], msg_sh.at[pl.ds(r0, ROWS_PER_TILE)])
    plsc.subcore_barrier()

    SUPER = 1024
    nsuper = E_PER_W // SUPER
    npair = SUPER // (2 * EB4)
    lanes = lax.iota(jnp.int32, 16)
    lane0 = (lanes == 0)

    def in_descs(e0, sub, exb, dnb, ehb, psb, sem_l, sem_g):
        e0s = pl.multiple_of(e0 + sub * EB4, 8)
        r8 = pl.multiple_of(e0s // 8, 8)
        return [pltpu.make_async_copy(ex_hbm.at[pl.ds(e0s, EB4)], exb, sem_l),
                pltpu.make_async_copy(eh_hbm.at[pl.ds(r8, EB4 // 8)], ehb, sem_l),
                pltpu.make_async_copy(den_hbm.at[didx_v.at[sub]], dnb, sem_g),
                pltpu.make_async_copy(ps_hbm.at[sidx_v.at[sub]], psb, sem_g)]

    def compute(exb, dnb, ehb, psb):
        def edge_t(i, _):
            exb[i] = exb[i] / dnb[i]
            return 0

        lax.fori_loop(0, EB4, edge_t, 0, unroll=8)

        def wgrp(g, _):
            rows = g * 16 + lanes
            acc = plsc.load_gather(exb, [rows, jnp.zeros((16,), jnp.int32)])
            for h in range(1, NH):
                acc = acc + plsc.load_gather(exb, [rows, jnp.full((16,), h, jnp.int32)])
            w_v[g] = acc * (1.0 / NH)
            return 0

        lax.fori_loop(0, EB4 // 16, wgrp, 0, unroll=4)

        def edge_v(i, _):
            wv = plsc.load_gather(
                w_v, [jnp.full((16,), i // 16, jnp.int32),
                      jnp.full((16,), i % 16, jnp.int32)])
            for j in range(D // 16):
                s = pl.ds(j * 16, 16)
                psb[i, s] = (ehb[i // 8, i % 8, s] + psb[i, s]) * wv
            return 0

        lax.fori_loop(0, EB4, edge_v, 0, unroll=2)

    def scat_descs(sub, psb):
        return [pltpu.make_async_copy(psb, msg_sh.at[didx_v.at[sub]], sem_s)]

    def chunk(c, _):
        e0 = pl.multiple_of(wid * E_PER_W + c * SUPER, 8)
        row0 = pl.multiple_of(e0 // EB4, 8)
        di0 = pltpu.async_copy(src2_hbm.at[pl.ds(row0, SUPER // EB4)], sidx_v, sem_i)
        di1 = pltpu.async_copy(dst2_hbm.at[pl.ds(row0, SUPER // EB4)], didx_v, sem_i)
        di0.wait()
        di1.wait()
        for d in in_descs(e0, 0, ex_a, dn_a, eh_a, ps_a, sem_la, sem_ga):
            d.start()
        for d in in_descs(e0, 1, ex_b, dn_b, eh_b, ps_b, sem_lb, sem_gb):
            d.start()

        def pair(p, _):
            sA = 2 * p
            sB = 2 * p + 1
            for d in in_descs(e0, sA, ex_a, dn_a, eh_a, ps_a, sem_la, sem_ga):
                d.wait()
            compute(ex_a, dn_a, eh_a, ps_a)
            dsA = scat_descs(sA, ps_a)
            for d in dsA:
                d.start(add=True)
            for d in in_descs(e0, sB, ex_b, dn_b, eh_b, ps_b, sem_lb, sem_gb):
                d.wait()
            for d in dsA:
                d.wait()

            @pl.when(p < npair - 1)
            def _():
                for d in in_descs(e0, sA + 2, ex_a, dn_a, eh_a, ps_a, sem_la, sem_ga):
                    d.start()

            compute(ex_b, dn_b, eh_b, ps_b)
            dsB = scat_descs(sB, ps_b)
            for d in dsB:
                d.start(add=True)
            for d in dsB:
                d.wait()

            @pl.when(p < npair - 1)
            def _():
                for d in in_descs(e0, sB + 2, ex_b, dn_b, eh_b, ps_b, sem_lb, sem_gb):
                    d.start()

            return 0

        lax.fori_loop(0, npair, pair, 0)
        return 0

    lax.fori_loop(0, nsuper, chunk, 0)

    plsc.subcore_barrier()
    pltpu.sync_copy(msg_sh.at[pl.ds(r0, ROWS_PER_TILE)],
                    msg_hbm.at[cid].at[pl.ds(r0, ROWS_PER_TILE)])


def kernel(node_feats, edge_feats, edge_index, W_trip, b_trip, W_score, b_score,
           W_self, b_self):
    f32 = jnp.float32
    weT = W_trip[:, :D].T
    wsT = W_trip[:, D:2 * D].T
    wdT = W_trip[:, 2 * D:].T
    wscT = jnp.concatenate([W_score, jnp.zeros((NHP - NH, D), f32)], axis=0).T
    bs16 = jnp.concatenate([b_score, jnp.zeros((NHP - NH,), f32)])

    nf_p = jnp.concatenate([node_feats, jnp.zeros((N_PAD - N_NODES, D), f32)], axis=0)
    src_p = jnp.concatenate([edge_index[0].astype(jnp.int32),
                             jnp.full((E_PAD - N_EDGES,), TRASH, jnp.int32)])
    dst_p = jnp.concatenate([edge_index[1].astype(jnp.int32),
                             jnp.full((E_PAD - N_EDGES,), TRASH, jnp.int32)])
    src2 = src_p.reshape(E_PAD // 128, 128)
    dst2 = dst_p.reshape(E_PAD // 128, 128)
    src2b = src_p.reshape(E_PAD // EB4, EB4)
    dst2b = dst_p.reshape(E_PAD // EB4, EB4)
    zden = jnp.zeros((N_PAD, NHP), f32)
    zmsg = jnp.zeros((N_PAD, D), f32)

    ps, pd_, ss, sd = pl.pallas_call(
        _k1_body,
        out_shape=(
            jax.ShapeDtypeStruct((N_PAD, D), f32),
            jax.ShapeDtypeStruct((N_PAD, D), f32),
            jax.ShapeDtypeStruct((N_PAD, NHP), f32),
            jax.ShapeDtypeStruct((N_PAD, NHP), f32),
        ),
    )(nf_p, wsT, wdT, wscT)

    EBLK = 2000
    grid = (N_EDGES // EBLK,)
    eh, se = pl.pallas_call(
        _k2_body,
        grid=grid,
        in_specs=[
            pl.BlockSpec((EBLK, D), lambda i: (i, 0)),
            pl.BlockSpec((D, D), lambda i: (0, 0)),
            pl.BlockSpec((D,), lambda i: (0,)),
            pl.BlockSpec((D, NHP), lambda i: (0, 0)),
            pl.BlockSpec((NHP,), lambda i: (0,)),
        ],
        out_specs=[
            pl.BlockSpec((EBLK, D), lambda i: (i, 0)),
            pl.BlockSpec((EBLK, NHP), lambda i: (i, 0)),
        ],
        out_shape=(
            jax.ShapeDtypeStruct((E_PAD, D), f32),
            jax.ShapeDtypeStruct((E_PAD, NHP), f32),
        ),
    )(edge_feats, weT, b_trip, wscT, bs16)

    mesh = plsc.VectorSubcoreMesh(core_axis_name="c", subcore_axis_name="s")
    sc_params = pltpu.CompilerParams(use_tc_tiling_on_sc=False,
                                     needs_layout_passes=False)
    k3 = functools.partial(
        pl.kernel, _k3_body, mesh=mesh, compiler_params=sc_params,
        out_type=(
            jax.ShapeDtypeStruct((E_PAD, NHP), f32),
            jax.ShapeDtypeStruct((NC, N_PAD, NHP), f32),
        ),
        scratch_types=[
            pltpu.VMEM((EB3 // 128, 128), jnp.int32),
            pltpu.VMEM((EB3 // 128, 128), jnp.int32),
            pltpu.VMEM((EB3, NHP), f32),
            pltpu.VMEM((EB3, NHP), f32),
            pltpu.VMEM((EB3, NHP), f32),
            pltpu.VMEM((EB3, NHP), f32),
            pltpu.VMEM_SHARED((N_PAD, NHP), f32),
            pltpu.SemaphoreType.DMA,
            pltpu.SemaphoreType.DMA,
            pltpu.SemaphoreType.DMA,
            pltpu.SemaphoreType.DMA,
        ],
    )()
    ex, den2 = k3(src2, dst2, se, ss, sd, zden)

    den = pl.pallas_call(
        _k3b_body,
        out_shape=jax.ShapeDtypeStruct((N_PAD, NHP), f32),
    )(den2[0], den2[1])

    eh3 = eh.reshape(E_PAD // 8, 8, D)
    k4 = functools.partial(
        pl.kernel, _k4_body, mesh=mesh, compiler_params=sc_params,
        out_type=jax.ShapeDtypeStruct((NC, N_PAD, D), f32),
        scratch_types=[
            pltpu.VMEM((1024 // EB4, EB4), jnp.int32),
            pltpu.VMEM((1024 // EB4, EB4), jnp.int32),
            pltpu.VMEM((EB4, NHP), f32),
            pltpu.VMEM((EB4, NHP), f32),
            pltpu.VMEM((EB4 // 8, 8, D), f32),
            pltpu.VMEM((EB4, D), f32),
            pltpu.VMEM((EB4, NHP), f32),
            pltpu.VMEM((EB4, NHP), f32),
            pltpu.VMEM((EB4 // 8, 8, D), f32),
            pltpu.VMEM((EB4, D), f32),
            pltpu.VMEM((EB4 // 16, 16), f32),
            pltpu.VMEM_SHARED((N_PAD, D), f32),
            pltpu.SemaphoreType.DMA,
            pltpu.SemaphoreType.DMA,
            pltpu.SemaphoreType.DMA,
            pltpu.SemaphoreType.DMA,
            pltpu.SemaphoreType.DMA,
            pltpu.SemaphoreType.DMA,
        ],
    )()
    msg = k4(src2b, dst2b, ex, den, eh3, ps, zmsg)

    out = pl.pallas_call(
        _k5_body,
        out_shape=jax.ShapeDtypeStruct((N_PAD, D), f32),
    )(nf_p, W_self.T, b_self, msg[0], msg[1], den, pd_)
    return out[:N_NODES]


# speedup vs baseline: 1.0207x; 1.0207x over previous
import functools

import jax
import jax.numpy as jnp
from jax import lax
from jax.experimental import pallas as pl
from jax.experimental.pallas import tpu as pltpu
from jax.experimental.pallas import tpu_sc as plsc

N_NODES = 10000
N_EDGES = 320000
D = 128
NH = 8
NHP = 16

NC = 2
NS = 16
NW = NC * NS

EB3 = 1024
EB4 = 64
E_PER_W = 10240
E_PAD = NW * E_PER_W
N_PAD = 10112
ROWS_PER_TILE = N_PAD // NS
TRASH = N_PAD - 1


def _k1_body(nf_ref, wsT_ref, wdT_ref, wscT_ref, ps_ref, pd_ref, ss_ref, sd_ref):
    nf = nf_ref[...]
    ps = jnp.dot(nf, wsT_ref[...], preferred_element_type=jnp.float32)
    pd = jnp.dot(nf, wdT_ref[...], preferred_element_type=jnp.float32)
    ps_ref[...] = ps
    pd_ref[...] = pd
    ss_ref[...] = jnp.dot(ps, wscT_ref[...], preferred_element_type=jnp.float32)
    sd_ref[...] = jnp.dot(pd, wscT_ref[...], preferred_element_type=jnp.float32)


def _k2_body(ef_ref, weT_ref, bt_ref, wscT_ref, bs_ref, eh_ref, se_ref):
    bf = jnp.bfloat16
    eh = jnp.dot(ef_ref[...].astype(bf), weT_ref[...].astype(bf),
                 preferred_element_type=jnp.float32)
    eh = eh + bt_ref[...]
    eh_ref[...] = eh
    se_ref[...] = jnp.dot(eh.astype(bf), wscT_ref[...].astype(bf),
                          preferred_element_type=jnp.float32) + bs_ref[...]


def _k3b_body(d0_ref, d1_ref, out_ref):
    out_ref[...] = d0_ref[...] + d1_ref[...]


def _k5_body(nf_ref, wselfT_ref, bself_ref, m0_ref, m1_ref, den_ref,
             pd_ref, out_ref):
    self_msg = jnp.dot(nf_ref[...], wselfT_ref[...], preferred_element_type=jnp.float32)
    c = jnp.where(den_ref[...][:, 0:1] > 0, 1.0, 0.0)
    out = m0_ref[...] + m1_ref[...] + c * pd_ref[...] + self_msg + bself_ref[...]
    slope = (1.0 / 8.0 + 1.0 / 3.0) / 2.0
    out_ref[...] = jnp.where(out >= 0, out, slope * out)


def _leaky(v):
    return jnp.where(v >= 0, v, 0.01 * v)


def _k3_body(src2_hbm, dst2_hbm, se_hbm, ss_hbm, sd_hbm, zden_hbm,
             ex_hbm, den_hbm,
             sidx_v, didx_v, se_v, ssg_v, sdg_v, ex_v, den_sh,
             sem_i, sem_l, sem_g, sem_s):
    cid = lax.axis_index("c")
    sid = lax.axis_index("s")
    wid = sid * NC + cid

    r0 = pl.multiple_of(sid * ROWS_PER_TILE, 8)
    pltpu.sync_copy(zden_hbm.at[pl.ds(r0, ROWS_PER_TILE)], den_sh.at[pl.ds(r0, ROWS_PER_TILE)])
    plsc.subcore_barrier()

    nchunks = E_PER_W // EB3
    ngrp = EB3 // 128

    def chunk(c, _):
        e0 = pl.multiple_of(wid * E_PER_W + c * EB3, 8)
        row0 = pl.multiple_of(e0 // 128, 8)
        di0 = pltpu.async_copy(src2_hbm.at[pl.ds(row0, ngrp)], sidx_v, sem_i)
        di1 = pltpu.async_copy(dst2_hbm.at[pl.ds(row0, ngrp)], didx_v, sem_i)
        dl = pltpu.async_copy(se_hbm.at[pl.ds(e0, EB3)], se_v, sem_l)
        di0.wait()
        di1.wait()
        gs = []
        for g in range(ngrp):
            gs.append(pltpu.async_copy(ss_hbm.at[sidx_v.at[g]],
                                       ssg_v.at[pl.ds(g * 128, 128)], sem_g))
            gs.append(pltpu.async_copy(sd_hbm.at[didx_v.at[g]],
                                       sdg_v.at[pl.ds(g * 128, 128)], sem_g))
        dl.wait()
        for d in gs:
            d.wait()

        def edge(i, _):
            v = se_v[i] + ssg_v[i] + sdg_v[i]
            ex_v[i] = jnp.exp(_leaky(v))
            return 0

        lax.fori_loop(0, EB3, edge, 0)
        ws = [pltpu.async_copy(ex_v, ex_hbm.at[pl.ds(e0, EB3)], sem_l)]
        for g in range(ngrp):
            ws.append(pltpu.async_copy(ex_v.at[pl.ds(g * 128, 128)],
                                       den_sh.at[didx_v.at[g]], sem_s, add=True))
        for d in ws:
            d.wait()
        return 0

    lax.fori_loop(0, nchunks, chunk, 0)

    plsc.subcore_barrier()
    pltpu.sync_copy(den_sh.at[pl.ds(r0, ROWS_PER_TILE)],
                    den_hbm.at[cid].at[pl.ds(r0, ROWS_PER_TILE)])


def _k4_body(src2_hbm, dst2_hbm, ex_hbm, den_hbm, eh_hbm, ps_hbm,
             zmsg_hbm,
             msg_hbm,
             sidx_v, didx_v, ex_a, dn_a, eh_a, ps_a, ex_b, dn_b, eh_b, ps_b,
             w_v, msg_sh,
             sem_i, sem_la, sem_ga, sem_lb, sem_gb, sem_s):
    cid = lax.axis_index("c")
    sid = lax.axis_index("s")
    wid = sid * NC + cid

    r0 = pl.multiple_of(sid * ROWS_PER_TILE, 8)
    pltpu.sync_copy(zmsg_hbm.at[pl.ds(r0, ROWS_PER_TILE)], msg_sh.at[pl.ds(r0, ROWS_PER_TILE)])
    plsc.subcore_barrier()

    SUPER = 1024
    nsuper = E_PER_W // SUPER
    npair = SUPER // (2 * EB4)
    lanes = lax.iota(jnp.int32, 16)
    lane0 = (lanes == 0)

    def in_descs(e0, sub, exb, dnb, ehb, psb, sem_l, sem_g):
        e0s = pl.multiple_of(e0 + sub * EB4, 8)
        r8 = pl.multiple_of(e0s // 8, 8)
        return [pltpu.make_async_copy(ex_hbm.at[pl.ds(e0s, EB4)], exb, sem_l),
                pltpu.make_async_copy(eh_hbm.at[pl.ds(r8, EB4 // 8)], ehb, sem_l),
                pltpu.make_async_copy(den_hbm.at[didx_v.at[sub]], dnb, sem_g),
                pltpu.make_async_copy(ps_hbm.at[sidx_v.at[sub]], psb, sem_g)]

    def compute(exb, dnb, ehb, psb):
        def edge_t(i, _):
            exb[i] = exb[i] / dnb[i]
            return 0

        lax.fori_loop(0, EB4, edge_t, 0, unroll=8)

        def wgrp(g, _):
            rows = g * 16 + lanes
            acc = plsc.load_gather(exb, [rows, jnp.zeros((16,), jnp.int32)])
            for h in range(1, NH):
                acc = acc + plsc.load_gather(exb, [rows, jnp.full((16,), h, jnp.int32)])
            w_v[g] = acc * (1.0 / NH)
            return 0

        lax.fori_loop(0, EB4 // 16, wgrp, 0, unroll=4)

        def edge_v(i, _):
            wv = plsc.load_gather(
                w_v, [jnp.full((16,), i // 16, jnp.int32),
                      jnp.full((16,), i % 16, jnp.int32)])
            for j in range(D // 16):
                s = pl.ds(j * 16, 16)
                psb[i, s] = (ehb[i // 8, i % 8, s] + psb[i, s]) * wv
            return 0

        lax.fori_loop(0, EB4, edge_v, 0, unroll=2)

    def scat_descs(sub, psb):
        return [pltpu.make_async_copy(psb, msg_sh.at[didx_v.at[sub]], sem_s)]

    def chunk(c, _):
        e0 = pl.multiple_of(wid * E_PER_W + c * SUPER, 8)
        row0 = pl.multiple_of(e0 // EB4, 8)
        di0 = pltpu.async_copy(src2_hbm.at[pl.ds(row0, SUPER // EB4)], sidx_v, sem_i)
        di1 = pltpu.async_copy(dst2_hbm.at[pl.ds(row0, SUPER // EB4)], didx_v, sem_i)
        di0.wait()
        di1.wait()
        for d in in_descs(e0, 0, ex_a, dn_a, eh_a, ps_a, sem_la, sem_ga):
            d.start()

        def pair(p, _):
            sA = 2 * p
            sB = 2 * p + 1
            for d in in_descs(e0, sA, ex_a, dn_a, eh_a, ps_a, sem_la, sem_ga):
                d.wait()
            for d in in_descs(e0, sB, ex_b, dn_b, eh_b, ps_b, sem_lb, sem_gb):
                d.start()
            compute(ex_a, dn_a, eh_a, ps_a)
            dsA = scat_descs(sA, ps_a)
            for d in dsA:
                d.start(add=True)
            for d in in_descs(e0, sB, ex_b, dn_b, eh_b, ps_b, sem_lb, sem_gb):
                d.wait()
            for d in dsA:
                d.wait()

            @pl.when(p < npair - 1)
            def _():
                for d in in_descs(e0, sA + 2, ex_a, dn_a, eh_a, ps_a, sem_la, sem_ga):
                    d.start()

            compute(ex_b, dn_b, eh_b, ps_b)
            dsB = scat_descs(sB, ps_b)
            for d in dsB:
                d.start(add=True)
            for d in dsB:
                d.wait()
            return 0

        lax.fori_loop(0, npair, pair, 0)
        return 0

    lax.fori_loop(0, nsuper, chunk, 0)

    plsc.subcore_barrier()
    pltpu.sync_copy(msg_sh.at[pl.ds(r0, ROWS_PER_TILE)],
                    msg_hbm.at[cid].at[pl.ds(r0, ROWS_PER_TILE)])


def kernel(node_feats, edge_feats, edge_index, W_trip, b_trip, W_score, b_score,
           W_self, b_self):
    f32 = jnp.float32
    weT = W_trip[:, :D].T
    wsT = W_trip[:, D:2 * D].T
    wdT = W_trip[:, 2 * D:].T
    wscT = jnp.concatenate([W_score, jnp.zeros((NHP - NH, D), f32)], axis=0).T
    bs16 = jnp.concatenate([b_score, jnp.zeros((NHP - NH,), f32)])

    nf_p = jnp.concatenate([node_feats, jnp.zeros((N_PAD - N_NODES, D), f32)], axis=0)
    src_p = jnp.concatenate([edge_index[0].astype(jnp.int32),
                             jnp.full((E_PAD - N_EDGES,), TRASH, jnp.int32)])
    dst_p = jnp.concatenate([edge_index[1].astype(jnp.int32),
                             jnp.full((E_PAD - N_EDGES,), TRASH, jnp.int32)])
    src2 = src_p.reshape(E_PAD // 128, 128)
    dst2 = dst_p.reshape(E_PAD // 128, 128)
    src2b = src_p.reshape(E_PAD // EB4, EB4)
    dst2b = dst_p.reshape(E_PAD // EB4, EB4)
    zden = jnp.zeros((N_PAD, NHP), f32)
    zmsg = jnp.zeros((N_PAD, D), f32)

    ps, pd_, ss, sd = pl.pallas_call(
        _k1_body,
        out_shape=(
            jax.ShapeDtypeStruct((N_PAD, D), f32),
            jax.ShapeDtypeStruct((N_PAD, D), f32),
            jax.ShapeDtypeStruct((N_PAD, NHP), f32),
            jax.ShapeDtypeStruct((N_PAD, NHP), f32),
        ),
    )(nf_p, wsT, wdT, wscT)

    EBLK = 2000
    grid = (N_EDGES // EBLK,)
    eh, se = pl.pallas_call(
        _k2_body,
        grid=grid,
        in_specs=[
            pl.BlockSpec((EBLK, D), lambda i: (i, 0)),
            pl.BlockSpec((D, D), lambda i: (0, 0)),
            pl.BlockSpec((D,), lambda i: (0,)),
            pl.BlockSpec((D, NHP), lambda i: (0, 0)),
            pl.BlockSpec((NHP,), lambda i: (0,)),
        ],
        out_specs=[
            pl.BlockSpec((EBLK, D), lambda i: (i, 0)),
            pl.BlockSpec((EBLK, NHP), lambda i: (i, 0)),
        ],
        out_shape=(
            jax.ShapeDtypeStruct((E_PAD, D), f32),
            jax.ShapeDtypeStruct((E_PAD, NHP), f32),
        ),
    )(edge_feats, weT, b_trip, wscT, bs16)

    mesh = plsc.VectorSubcoreMesh(core_axis_name="c", subcore_axis_name="s")
    sc_params = pltpu.CompilerParams(use_tc_tiling_on_sc=False,
                                     needs_layout_passes=False)
    k3 = functools.partial(
        pl.kernel, _k3_body, mesh=mesh, compiler_params=sc_params,
        out_type=(
            jax.ShapeDtypeStruct((E_PAD, NHP), f32),
            jax.ShapeDtypeStruct((NC, N_PAD, NHP), f32),
        ),
        scratch_types=[
            pltpu.VMEM((EB3 // 128, 128), jnp.int32),
            pltpu.VMEM((EB3 // 128, 128), jnp.int32),
            pltpu.VMEM((EB3, NHP), f32),
            pltpu.VMEM((EB3, NHP), f32),
            pltpu.VMEM((EB3, NHP), f32),
            pltpu.VMEM((EB3, NHP), f32),
            pltpu.VMEM_SHARED((N_PAD, NHP), f32),
            pltpu.SemaphoreType.DMA,
            pltpu.SemaphoreType.DMA,
            pltpu.SemaphoreType.DMA,
            pltpu.SemaphoreType.DMA,
        ],
    )()
    ex, den2 = k3(src2, dst2, se, ss, sd, zden)

    den = pl.pallas_call(
        _k3b_body,
        out_shape=jax.ShapeDtypeStruct((N_PAD, NHP), f32),
    )(den2[0], den2[1])

    eh3 = eh.reshape(E_PAD // 8, 8, D)
    k4 = functools.partial(
        pl.kernel, _k4_body, mesh=mesh, compiler_params=sc_params,
        out_type=jax.ShapeDtypeStruct((NC, N_PAD, D), f32),
        scratch_types=[
            pltpu.VMEM((1024 // EB4, EB4), jnp.int32),
            pltpu.VMEM((1024 // EB4, EB4), jnp.int32),
            pltpu.VMEM((EB4, NHP), f32),
            pltpu.VMEM((EB4, NHP), f32),
            pltpu.VMEM((EB4 // 8, 8, D), f32),
            pltpu.VMEM((EB4, D), f32),
            pltpu.VMEM((EB4, NHP), f32),
            pltpu.VMEM((EB4, NHP), f32),
            pltpu.VMEM((EB4 // 8, 8, D), f32),
            pltpu.VMEM((EB4, D), f32),
            pltpu.VMEM((EB4 // 16, 16), f32),
            pltpu.VMEM_SHARED((N_PAD, D), f32),
            pltpu.SemaphoreType.DMA,
            pltpu.SemaphoreType.DMA,
            pltpu.SemaphoreType.DMA,
            pltpu.SemaphoreType.DMA,
            pltpu.SemaphoreType.DMA,
            pltpu.SemaphoreType.DMA,
        ],
    )()
    msg = k4(src2b, dst2b, ex, den, eh3, ps, zmsg)

    out = pl.pallas_call(
        _k5_body,
        out_shape=jax.ShapeDtypeStruct((N_PAD, D), f32),
    )(nf_p, W_self.T, b_self, msg[0], msg[1], den, pd_)
    return out[:N_NODES]
